# baseline (speedup 1.0000x reference)
import functools

import jax
import jax.numpy as jnp
from jax import lax
from jax.experimental import pallas as pl
from jax.experimental.pallas import tpu as pltpu
from jax.experimental.pallas import tpu_sc as plsc

B, C, H, W = 8, 32, 32, 32
N = B * H * W
K = 8192
BLK = 256
GRID = N // BLK
CHUNK = 2048
NCHUNK = K // CHUNK


def _vq_tc_kernel(flatt_ref, ft2bf_ref, cbbf_ref, csum_ref, idx_ref, dmin_ref):
    ft = flatt_ref[...]
    sq = ft * ft
    a = sq[0:1, :]
    for c in range(1, 32):
        a = a + sq[c:c + 1, :]

    ft2_bf = ft2bf_ref[...]
    cb_bf = cbbf_ref[...]
    kio = lax.broadcasted_iota(jnp.int32, (CHUNK, BLK), 0)
    big = jnp.int32(2**31 - 1)

    carry_v = None
    for ch in range(NCHUNK):
        s2 = lax.dot_general(cb_bf[ch * CHUNK:(ch + 1) * CHUNK, :], ft2_bf,
                             dimension_numbers=(((1,), (0,)), ((), ())),
                             preferred_element_type=jnp.float32)
        d = (a - s2) + csum_ref[ch * CHUNK:(ch + 1) * CHUNK, :]
        m = jnp.min(d, axis=0, keepdims=True)
        mi = jnp.min(jnp.where(d == m, kio, big),
                     axis=0, keepdims=True) + jnp.int32(ch * CHUNK)
        if carry_v is None:
            carry_v = m.astype(jnp.bfloat16)
            sel_v, sel_i = m, mi
        else:
            up = carry_v.astype(jnp.float32)
            take = m < up
            carry_v = jnp.where(take, m, up).astype(jnp.bfloat16)
            sel_v = jnp.where(take, m, sel_v)
            sel_i = jnp.where(take, mi, sel_i)

    idx_ref[...] = sel_i.reshape(1, 1, BLK)
    dmin_ref[...] = sel_v.reshape(1, 1, BLK)


def _vq_argmin(flatt, ft2bf, cbbf, csum):
    return pl.pallas_call(
        _vq_tc_kernel,
        grid=(GRID,),
        in_specs=[
            pl.BlockSpec((C, BLK), lambda i: (0, i)),
            pl.BlockSpec((C, BLK), lambda i: (0, i)),
            pl.BlockSpec((K, C), lambda i: (0, 0)),
            pl.BlockSpec((K, 1), lambda i: (0, 0)),
        ],
        out_specs=[
            pl.BlockSpec((1, 1, BLK), lambda i: (i, 0, 0)),
            pl.BlockSpec((1, 1, BLK), lambda i: (i, 0, 0)),
        ],
        out_shape=[
            jax.ShapeDtypeStruct((GRID, 1, BLK), jnp.int32),
            jax.ShapeDtypeStruct((GRID, 1, BLK), jnp.float32),
        ],
    )(flatt, ft2bf, cbbf, csum)


_SC_NC = 2
_SC_NS = 16
_NW = _SC_NC * _SC_NS
_BPW = N // _NW


def _sc_gather(codebook, idx):
    mesh = plsc.VectorSubcoreMesh(core_axis_name="c", subcore_axis_name="s")

    @functools.partial(
        pl.kernel, mesh=mesh,
        compiler_params=pltpu.CompilerParams(use_tc_tiling_on_sc=False),
        out_type=jax.ShapeDtypeStruct((N, C), jnp.float32),
        scratch_types=[
            pltpu.VMEM((_BPW,), jnp.int32),
            pltpu.VMEM((_BPW, C), jnp.float32),
            pltpu.SemaphoreType.DMA,
        ],
    )
    def k(table_hbm, idx_hbm, out_hbm, idx_v, rows_v, sem):
        wid = lax.axis_index("s") * _SC_NC + lax.axis_index("c")
        base = wid * _BPW
        pltpu.sync_copy(idx_hbm.at[pl.ds(base, _BPW)], idx_v)
        pltpu.async_copy(table_hbm.at[idx_v], rows_v, sem).wait()
        pltpu.sync_copy(rows_v, out_hbm.at[pl.ds(base, _BPW)])

    return k(codebook, idx)


def kernel(x, codebook):
    flatt = x.reshape(B, C, H * W).transpose(1, 0, 2).reshape(C, N)
    ft2bf = (2.0 * flatt).astype(jnp.bfloat16)
    cbbf = codebook.astype(jnp.bfloat16)
    csum = jnp.sum(codebook * codebook, axis=1, keepdims=True)
    idx3, dmin3 = _vq_argmin(flatt, ft2bf, cbbf, csum)
    idx = idx3.reshape(N)
    quant = _sc_gather(codebook, idx)
    x_rec = quant.reshape(B, H, W, C).transpose(0, 3, 1, 2)
    embedding_loss = (jnp.sum(dmin3) / jnp.float32(N * C)).astype(jnp.float32)
    commit_loss = jnp.float32(0.25) * embedding_loss
    return (x_rec, embedding_loss, commit_loss)


# speedup vs baseline: 1.0906x; 1.0906x over previous
import functools

import jax
import jax.numpy as jnp
from jax import lax
from jax.experimental import pallas as pl
from jax.experimental.pallas import tpu as pltpu
from jax.experimental.pallas import tpu_sc as plsc

B, C, H, W = 8, 32, 32, 32
N = B * H * W
K = 8192
BLK = 512
GRID = N // BLK
CHUNK = 2048
NCHUNK = K // CHUNK


def _vq_tc_kernel(flatt_ref, ft2bf_ref, cbbf_ref, csum_ref, idx_ref, dmin_ref):
    ft = flatt_ref[...]
    sq = ft * ft
    a = sq[0:1, :]
    for c in range(1, 32):
        a = a + sq[c:c + 1, :]

    ft2_bf = ft2bf_ref[...]
    cb_bf = cbbf_ref[...]
    kio = lax.broadcasted_iota(jnp.int32, (CHUNK, BLK), 0)
    big = jnp.int32(2**31 - 1)

    carry_v = None
    for ch in range(NCHUNK):
        s2 = lax.dot_general(cb_bf[ch * CHUNK:(ch + 1) * CHUNK, :], ft2_bf,
                             dimension_numbers=(((1,), (0,)), ((), ())),
                             preferred_element_type=jnp.float32)
        d = (a - s2) + csum_ref[ch * CHUNK:(ch + 1) * CHUNK, :]
        m = jnp.min(d, axis=0, keepdims=True)
        mi = jnp.min(jnp.where(d == m, kio, big),
                     axis=0, keepdims=True) + jnp.int32(ch * CHUNK)
        if carry_v is None:
            carry_v = m.astype(jnp.bfloat16)
            sel_v, sel_i = m, mi
        else:
            up = carry_v.astype(jnp.float32)
            take = m < up
            carry_v = jnp.where(take, m, up).astype(jnp.bfloat16)
            sel_v = jnp.where(take, m, sel_v)
            sel_i = jnp.where(take, mi, sel_i)

    idx_ref[...] = sel_i.reshape(1, 1, BLK)
    dmin_ref[...] = sel_v.reshape(1, 1, BLK)


def _vq_argmin(flatt, ft2bf, cbbf, csum):
    return pl.pallas_call(
        _vq_tc_kernel,
        grid=(GRID,),
        in_specs=[
            pl.BlockSpec((C, BLK), lambda i: (0, i)),
            pl.BlockSpec((C, BLK), lambda i: (0, i)),
            pl.BlockSpec((K, C), lambda i: (0, 0)),
            pl.BlockSpec((K, 1), lambda i: (0, 0)),
        ],
        out_specs=[
            pl.BlockSpec((1, 1, BLK), lambda i: (i, 0, 0)),
            pl.BlockSpec((1, 1, BLK), lambda i: (i, 0, 0)),
        ],
        out_shape=[
            jax.ShapeDtypeStruct((GRID, 1, BLK), jnp.int32),
            jax.ShapeDtypeStruct((GRID, 1, BLK), jnp.float32),
        ],
    )(flatt, ft2bf, cbbf, csum)


_SC_NC = 2
_SC_NS = 16
_NW = _SC_NC * _SC_NS
_BPW = N // _NW


def _sc_gather(codebook, idx):
    mesh = plsc.VectorSubcoreMesh(core_axis_name="c", subcore_axis_name="s")

    @functools.partial(
        pl.kernel, mesh=mesh,
        compiler_params=pltpu.CompilerParams(use_tc_tiling_on_sc=False),
        out_type=jax.ShapeDtypeStruct((N, C), jnp.float32),
        scratch_types=[
            pltpu.VMEM((_BPW,), jnp.int32),
            pltpu.VMEM((_BPW, C), jnp.float32),
            pltpu.SemaphoreType.DMA,
        ],
    )
    def k(table_hbm, idx_hbm, out_hbm, idx_v, rows_v, sem):
        wid = lax.axis_index("s") * _SC_NC + lax.axis_index("c")
        base = wid * _BPW
        pltpu.sync_copy(idx_hbm.at[pl.ds(base, _BPW)], idx_v)
        pltpu.async_copy(table_hbm.at[idx_v], rows_v, sem).wait()
        pltpu.sync_copy(rows_v, out_hbm.at[pl.ds(base, _BPW)])

    return k(codebook, idx)


def kernel(x, codebook):
    flatt = x.reshape(B, C, H * W).transpose(1, 0, 2).reshape(C, N)
    ft2bf = (2.0 * flatt).astype(jnp.bfloat16)
    cbbf = codebook.astype(jnp.bfloat16)
    csum = jnp.sum(codebook * codebook, axis=1, keepdims=True)
    idx3, dmin3 = _vq_argmin(flatt, ft2bf, cbbf, csum)
    idx = idx3.reshape(N)
    quant = _sc_gather(codebook, idx)
    x_rec = quant.reshape(B, H, W, C).transpose(0, 3, 1, 2)
    embedding_loss = (jnp.sum(dmin3) / jnp.float32(N * C)).astype(jnp.float32)
    commit_loss = jnp.float32(0.25) * embedding_loss
    return (x_rec, embedding_loss, commit_loss)
